# Initial kernel scaffold; baseline (speedup 1.0000x reference)
import jax
import jax.numpy as jnp
from jax.experimental import pallas as pl


def kernel(keys, query, value):
    raise NotImplementedError("write your pallas kernel here")


# speedup vs baseline: 5.4325x; 5.4325x over previous
import jax
import jax.numpy as jnp
from jax import lax
from jax.experimental import pallas as pl
from jax.experimental.pallas import tpu as pltpu

M = 2048
D = 128
N = 8192
TN = 512
NB = N // TN


def _body(q_ref, v_ref, k_ref, out_ref, rowmax, rowarg, colmax, acc):
    p = pl.program_id(0)
    b = pl.program_id(1)

    @pl.when(p == 0)
    def _phase0():
        q = q_ref[...]
        ss = jnp.sum(q * q, axis=1, keepdims=True)
        qn = q / jnp.maximum(jnp.sqrt(ss), 1e-12)
        score = lax.dot_general(
            qn, k_ref[...], (((1,), (1,)), ((), ())),
            preferred_element_type=jnp.float32)
        smax = jnp.max(score, axis=1, keepdims=True)
        iota = lax.broadcasted_iota(jnp.int32, (TN, M), 1)
        g = jnp.min(jnp.where(score == smax, iota, M), axis=1)
        rowmax[b, :] = smax[:, 0]
        rowarg[b, :] = g
        pc = jnp.max(score, axis=0, keepdims=True)

        @pl.when(b == 0)
        def _():
            colmax[...] = pc
            acc[...] = jnp.zeros_like(acc)

        @pl.when(b > 0)
        def _():
            colmax[...] = jnp.maximum(colmax[...], pc)

    @pl.when(p == 1)
    def _phase1():
        s = rowmax[b, :][:, None]
        g = rowarg[b, :][:, None]
        iota = lax.broadcasted_iota(jnp.int32, (TN, M), 1)
        onehot = iota == g
        cg = jnp.max(jnp.where(onehot, colmax[...], -jnp.inf),
                     axis=1, keepdims=True)
        w = jnp.exp(s - cg)
        woh = jnp.where(onehot, w, 0.0)
        v = v_ref[...]
        acc[...] += lax.dot_general(
            woh, v, (((0,), (0,)), ((), ())),
            preferred_element_type=jnp.float32)

        @pl.when(b == NB - 1)
        def _():
            mem = acc[...] + k_ref[...]
            nn = jnp.sqrt(jnp.sum(mem * mem, axis=1, keepdims=True))
            out_ref[...] = mem / jnp.maximum(nn, 1e-12)


def kernel(keys, query, value):
    b_, d_, h_, w_ = query.shape
    qf = jnp.transpose(query, (0, 2, 3, 1)).reshape(N, D)
    vf = jnp.transpose(value, (0, 2, 3, 1)).reshape(N, D)

    out = pl.pallas_call(
        _body,
        grid=(2, NB),
        in_specs=[
            pl.BlockSpec((TN, D), lambda p, b: (jnp.where(p == 0, b, 0), 0)),
            pl.BlockSpec((TN, D), lambda p, b: (jnp.where(p == 0, 0, b), 0)),
            pl.BlockSpec((M, D), lambda p, b: (0, 0)),
        ],
        out_specs=pl.BlockSpec((M, D), lambda p, b: (0, 0)),
        out_shape=jax.ShapeDtypeStruct((M, D), jnp.float32),
        scratch_shapes=[
            pltpu.VMEM((NB, TN), jnp.float32),
            pltpu.VMEM((NB, TN), jnp.int32),
            pltpu.VMEM((1, M), jnp.float32),
            pltpu.VMEM((M, D), jnp.float32),
        ],
    )(qf, vf, keys)
    return out


# speedup vs baseline: 5.9645x; 1.0979x over previous
import jax
import jax.numpy as jnp
from jax import lax
from jax.experimental import pallas as pl
from jax.experimental.pallas import tpu as pltpu

M = 2048
D = 128
N = 8192
TN = 512
NB = N // TN


def _body(q_ref, v_ref, k_ref, out_ref, rowmax, rowarg, colmax, cmacc, acc):
    p = pl.program_id(0)
    b = pl.program_id(1)

    @pl.when(p == 0)
    def _phase0():
        q = q_ref[...]
        ss = jnp.sum(q * q, axis=1, keepdims=True)
        qn = q / jnp.maximum(jnp.sqrt(ss), 1e-12)
        score = lax.dot_general(
            k_ref[...], qn, (((1,), (1,)), ((), ())),
            preferred_element_type=jnp.float32)
        smax = jnp.max(score, axis=0, keepdims=True)
        iota0 = lax.broadcasted_iota(jnp.int32, (M, TN), 0)
        g = jnp.min(jnp.where(score == smax, iota0, M), axis=0)
        rowmax[b, :] = smax[0, :]
        rowarg[b, :] = g

        @pl.when(b == 0)
        def _():
            cmacc[...] = score
            acc[...] = jnp.zeros_like(acc)

        @pl.when(b > 0)
        def _():
            cmacc[...] = jnp.maximum(cmacc[...], score)

        @pl.when(b == NB - 1)
        def _():
            colmax[...] = jnp.max(cmacc[...], axis=1, keepdims=True)

    @pl.when(p == 1)
    def _phase1():
        s = rowmax[b, :][None, :]
        g = rowarg[b, :][None, :]
        iota0 = lax.broadcasted_iota(jnp.int32, (M, TN), 0)
        onehot = iota0 == g
        cg = jnp.max(jnp.where(onehot, colmax[...], -jnp.inf),
                     axis=0, keepdims=True)
        w = jnp.exp(s - cg)
        woh = jnp.where(onehot, w, 0.0)
        acc[...] += lax.dot_general(
            woh, v_ref[...], (((1,), (0,)), ((), ())),
            preferred_element_type=jnp.float32)

        @pl.when(b == NB - 1)
        def _():
            mem = acc[...] + k_ref[...]
            nn = jnp.sqrt(jnp.sum(mem * mem, axis=1, keepdims=True))
            out_ref[...] = mem / jnp.maximum(nn, 1e-12)


def kernel(keys, query, value):
    b_, d_, h_, w_ = query.shape
    qf = jnp.transpose(query, (0, 2, 3, 1)).reshape(N, D)
    vf = jnp.transpose(value, (0, 2, 3, 1)).reshape(N, D)

    out = pl.pallas_call(
        _body,
        grid=(2, NB),
        in_specs=[
            pl.BlockSpec((TN, D), lambda p, b: (jnp.where(p == 0, b, 0), 0)),
            pl.BlockSpec((TN, D), lambda p, b: (jnp.where(p == 0, 0, b), 0)),
            pl.BlockSpec((M, D), lambda p, b: (0, 0)),
        ],
        out_specs=pl.BlockSpec((M, D), lambda p, b: (0, 0)),
        out_shape=jax.ShapeDtypeStruct((M, D), jnp.float32),
        scratch_shapes=[
            pltpu.VMEM((NB, TN), jnp.float32),
            pltpu.VMEM((NB, TN), jnp.int32),
            pltpu.VMEM((M, 1), jnp.float32),
            pltpu.VMEM((M, TN), jnp.float32),
            pltpu.VMEM((M, D), jnp.float32),
        ],
    )(qf, vf, keys)
    return out
